# baseline (speedup 1.0000x reference)
import jax
import jax.numpy as jnp
from jax import lax
from jax.experimental import pallas as pl
from jax.experimental.pallas import tpu as pltpu
from jax.experimental.pallas import tpu_sc as plsc

C = 100000
D = 64
B = 16384
LAMB = 1.0
ALPHA = 0.5

NC = 2
NS = 16
HALF = C // NC
CHUNK = 10000
NCHUNK = 5
SPT = B // NS
BLK = 128
NBLK = SPT // BLK
RPT = 632
ACC_ROWS = RPT * NS
ZQ = 784
ZREP = 4
CNT_LEN = ZQ * ZREP * NS
DBLK = 128
NDB = (RPT + DBLK - 1) // DBLK


def _body(feat_hbm, tgt_hbm, ctr_hbm, loss_hbm, out_hbm,
          idx_v, cidx_r, cntb, ones_v, zbuf, fbuf, cbuf,
          vbuf, lrow, lall, lout, counts_sh, accum_sh, loss_sh):
    c = lax.axis_index("c")
    s = lax.axis_index("s")
    cbase = c * HALF

    pltpu.sync_copy(tgt_hbm.at[pl.ds(s * SPT, SPT)], idx_v)

    def _fill_ones(i, x):
        ones_v[pl.ds(i * 16, 16)] = jnp.ones((16,), jnp.float32)
        return x
    lax.fori_loop(0, BLK // 16, _fill_ones, 0)

    def _fill_z(i, x):
        zbuf[pl.ds(i * 16, 16)] = jnp.zeros((16,), jnp.float32)
        return x
    lax.fori_loop(0, ZQ // 16, _fill_z, 0)

    for j in range(NBLK):
        for u in range(BLK // 16):
            t = idx_v[pl.ds(j * BLK + u * 16, 16)]
            tl = t - cbase
            inh = jnp.logical_and(tl >= 0, tl < HALF)
            cidx_r[j, pl.ds(u * 16, 16)] = jnp.where(inh, tl, HALF)

    for zr in range(ZREP):
        pltpu.sync_copy(zbuf, counts_sh.at[pl.ds((s * ZREP + zr) * ZQ, ZQ)])
    plsc.subcore_barrier()
    for j in range(NBLK):
        pltpu.sync_copy(ones_v, counts_sh.at[cidx_r.at[j]], add=True)
    plsc.subcore_barrier()

    def _blk(b, lacc):
        base = b * BLK
        pltpu.sync_copy(feat_hbm.at[pl.ds(s * SPT + base, BLK)], fbuf)
        pltpu.sync_copy(ctr_hbm.at[idx_v.at[pl.ds(base, BLK)]], cbuf)
        pltpu.sync_copy(counts_sh.at[cidx_r.at[b]], cntb)

        def _grp(g, lacc):
            cnt16 = cntb[pl.ds(g * 16, 16)]
            s16 = ALPHA / (cnt16 + 1.0)
            for j in range(16):
                sj = jnp.broadcast_to(s16[j], (16,))
                r = g * 16 + j
                for f in range(D // 16):
                    cv = cbuf[r, pl.ds(f * 16, 16)]
                    fv = fbuf[r, pl.ds(f * 16, 16)]
                    d = cv - fv
                    vbuf[base + r, pl.ds(f * 16, 16)] = d * sj
                    lacc = lacc + d * d
            return lacc
        return lax.fori_loop(0, BLK // 16, _grp, lacc)

    lacc = lax.fori_loop(0, NBLK, _blk, jnp.zeros((16,), jnp.float32))

    lrow[...] = lacc
    pltpu.sync_copy(lrow, loss_sh.at[pl.ds(s * 16, 16)])

    for k in range(NCHUNK):
        lo = k * CHUNK
        crows = min(CHUNK, HALF - lo)
        def _zf(r, x):
            for f in range(D // 16):
                fbuf[r, pl.ds(f * 16, 16)] = jnp.zeros((16,), jnp.float32)
            return x
        lax.fori_loop(0, DBLK, _zf, 0)
        start = s * RPT
        for i in range(NDB):
            bb = jnp.minimum(start + i * DBLK, start + RPT - DBLK)
            pltpu.sync_copy(fbuf, accum_sh.at[pl.ds(bb, DBLK)])
        plsc.subcore_barrier()

        for j in range(NBLK):
            for u in range(BLK // 16):
                t = idx_v[pl.ds(j * BLK + u * 16, 16)]
                tl = t - cbase - lo
                ink = jnp.logical_and(tl >= 0, tl < CHUNK)
                cidx_r[j, pl.ds(u * 16, 16)] = jnp.where(ink, tl, CHUNK)
        for j in range(NBLK):
            pltpu.sync_copy(vbuf.at[pl.ds(j * BLK, BLK)],
                            accum_sh.at[cidx_r.at[j]], add=True)
        plsc.subcore_barrier()

        rows0 = cbase + lo
        dstart = s * RPT
        dend = jnp.minimum(dstart + RPT, crows)

        def _dblk(i, x):
            bb = jnp.minimum(dstart + i * DBLK, dend - DBLK)
            pltpu.sync_copy(ctr_hbm.at[pl.ds(rows0 + bb, DBLK)], fbuf)
            pltpu.sync_copy(accum_sh.at[pl.ds(bb, DBLK)], cbuf)

            def _row(r, y):
                for f in range(D // 16):
                    fbuf[r, pl.ds(f * 16, 16)] = (
                        fbuf[r, pl.ds(f * 16, 16)] - cbuf[r, pl.ds(f * 16, 16)])
                return y
            lax.fori_loop(0, DBLK, _row, 0)
            pltpu.sync_copy(fbuf, out_hbm.at[pl.ds(rows0 + bb, DBLK)])
            return x
        nb = (dend - dstart + DBLK - 1) // DBLK
        lax.fori_loop(0, nb, _dblk, 0)
        plsc.subcore_barrier()

    @pl.when(jnp.logical_and(c == 0, s == 0))
    def _():
        pltpu.sync_copy(loss_sh, lall)
        acc = jnp.zeros((16,), jnp.float32)
        for r in range(NS):
            acc = acc + lall[pl.ds(r * 16, 16)]
        lout[...] = acc * (LAMB / float(B * D))
        pltpu.sync_copy(lout, loss_hbm)


_mesh = plsc.VectorSubcoreMesh(core_axis_name="c", subcore_axis_name="s",
                               num_cores=NC, num_subcores=NS)

_sc_call = pl.kernel(
    _body,
    out_type=(jax.ShapeDtypeStruct((16,), jnp.float32),
              jax.ShapeDtypeStruct((C, D), jnp.float32)),
    mesh=_mesh,
    compiler_params=pltpu.CompilerParams(use_tc_tiling_on_sc=False),
    scratch_types=(
        pltpu.VMEM((SPT,), jnp.int32),
        pltpu.VMEM((NBLK, BLK), jnp.int32),
        pltpu.VMEM((BLK,), jnp.float32),
        pltpu.VMEM((BLK,), jnp.float32),
        pltpu.VMEM((ZQ,), jnp.float32),
        pltpu.VMEM((DBLK, D), jnp.float32),
        pltpu.VMEM((DBLK, D), jnp.float32),
        pltpu.VMEM((SPT, D), jnp.float32),
        pltpu.VMEM((16,), jnp.float32),
        pltpu.VMEM((NS * 16,), jnp.float32),
        pltpu.VMEM((16,), jnp.float32),
        pltpu.MemorySpace.VMEM_SHARED((CNT_LEN,), jnp.float32),
        pltpu.MemorySpace.VMEM_SHARED((ACC_ROWS, D), jnp.float32),
        pltpu.MemorySpace.VMEM_SHARED((NS * 16,), jnp.float32),
    ),
)


@jax.jit
def kernel(features, targets, centers):
    loss_v, new_centers = _sc_call(features, targets, centers)
    return jnp.sum(loss_v), new_centers


# speedup vs baseline: 1.2470x; 1.2470x over previous
import jax
import jax.numpy as jnp
from jax import lax
from jax.experimental import pallas as pl
from jax.experimental.pallas import tpu as pltpu
from jax.experimental.pallas import tpu_sc as plsc

C = 100000
D = 64
B = 16384
LAMB = 1.0
ALPHA = 0.5

NC = 2
NS = 16
HALF = C // NC
CHUNK = 10000
NCHUNK = 5
SPT = B // NS
BLK = 128
NBLK = SPT // BLK
PBLK = 32
NPB = SPT // PBLK
RPT = 624
TAIL0 = RPT * NS
TAILN = CHUNK - TAIL0
ACC_ROWS = 10112
ZQ = 784
ZREP = 4
CNT_LEN = ZQ * ZREP * NS


def _body(feat_hbm, tgt_hbm, ctr_hbm, loss_hbm, out_hbm,
          idx_v, cidx_r, cidx_l, ones_v, zbuf,
          fb_a, cb_a, nb_a, fb_b, cb_b, nb_b,
          vbuf, lrow, lout, counts_sh, accum_sh, loss_sh,
          sfa, sca, sna, sfb, scb, snb, ssc, sso):
    c = lax.axis_index("c")
    s = lax.axis_index("s")
    cbase = c * HALF

    pltpu.sync_copy(tgt_hbm.at[pl.ds(s * SPT, SPT)], idx_v)

    def _fill_ones(i, x):
        ones_v[pl.ds(i * 16, 16)] = jnp.ones((16,), jnp.float32)
        return x
    lax.fori_loop(0, BLK // 16, _fill_ones, 0)

    def _fill_z(i, x):
        zbuf[pl.ds(i * 16, 16)] = jnp.zeros((16,), jnp.float32)
        return x
    lax.fori_loop(0, ZQ // 16, _fill_z, 0)

    def _cidx(j, x):
        for u in range(BLK // 16):
            t = idx_v[pl.ds(j * BLK + u * 16, 16)]
            tl = t - cbase
            inh = jnp.logical_and(tl >= 0, tl < HALF)
            ci = jnp.where(inh, tl, HALF + (t & 63))
            cidx_r[j, pl.ds(u * 16, 16)] = ci
            cidx_l[pl.ds(j * BLK + u * 16, 16)] = ci
        return x
    lax.fori_loop(0, NBLK, _cidx, 0)
    plsc.subcore_barrier()

    for zr in range(ZREP):
        pltpu.sync_copy(zbuf, counts_sh.at[pl.ds((s * ZREP + zr) * ZQ, ZQ)])
    plsc.subcore_barrier()
    for j in range(NBLK):
        pltpu.sync_copy(ones_v, counts_sh.at[cidx_r.at[j]], add=True)
    plsc.subcore_barrier()

    def _feat_slice(b):
        return feat_hbm.at[pl.ds(s * SPT + b * PBLK, PBLK)]

    def _gat_slice(b):
        return ctr_hbm.at[idx_v.at[pl.ds(b * PBLK, PBLK)]]

    def _cnt_slice(b):
        return counts_sh.at[cidx_l.at[pl.ds(b * PBLK, PBLK)]]

    def _compute(b, fb, cb, nb, lacc):
        vb = b * PBLK

        def _grp(g, lacc):
            cnt16 = nb[pl.ds(g * 16, 16)]
            s16 = ALPHA / (cnt16 + 1.0)
            for j in range(16):
                sj = jnp.broadcast_to(s16[j], (16,))
                r = g * 16 + j
                for f in range(D // 16):
                    fv = fb[r, pl.ds(f * 16, 16)]
                    cv = cb[r, pl.ds(f * 16, 16)]
                    d = fv - cv
                    vbuf[vb + r, pl.ds(f * 16, 16)] = d * sj
                    lacc = lacc + d * d
            return lacc
        return lax.fori_loop(0, PBLK // 16, _grp, lacc)

    def _sblk(b, lacc):
        pltpu.sync_copy(_feat_slice(b), fb_a)
        pltpu.sync_copy(_gat_slice(b), cb_a)
        pltpu.sync_copy(_cnt_slice(b), nb_a)
        return _compute(b, fb_a, cb_a, nb_a, lacc)

    lacc = lax.fori_loop(0, NPB, _sblk, jnp.zeros((16,), jnp.float32))

    lrow[...] = lacc
    plsc.subcore_barrier()
    pltpu.sync_copy(lrow, loss_sh.at[pl.ds(s * 16, 16)])

    dstart = s * RPT

    for k in range(NCHUNK):
        lo = k * CHUNK
        pltpu.sync_copy(ctr_hbm.at[pl.ds(cbase + lo + dstart, RPT)],
                        accum_sh.at[pl.ds(dstart, RPT)])

        @pl.when(s == NS - 1)
        def _t():
            pltpu.sync_copy(ctr_hbm.at[pl.ds(cbase + lo + TAIL0, TAILN)],
                            accum_sh.at[pl.ds(TAIL0, TAILN)])
        plsc.subcore_barrier()

        def _kidx(j, x):
            for u in range(BLK // 16):
                t = idx_v[pl.ds(j * BLK + u * 16, 16)]
                tl = t - cbase - lo
                ink = jnp.logical_and(tl >= 0, tl < CHUNK)
                cidx_r[j, pl.ds(u * 16, 16)] = jnp.where(ink, tl,
                                                         CHUNK + (t & 63))
            return x
        lax.fori_loop(0, NBLK, _kidx, 0)
        plsc.subcore_barrier()

        for j in range(NBLK):
            pltpu.sync_copy(vbuf.at[pl.ds(j * BLK, BLK)],
                            accum_sh.at[cidx_r.at[j]], add=True)
        plsc.subcore_barrier()

        pltpu.sync_copy(accum_sh.at[pl.ds(dstart, RPT)],
                        out_hbm.at[pl.ds(cbase + lo + dstart, RPT)])

        @pl.when(s == NS - 1)
        def _t2():
            pltpu.sync_copy(accum_sh.at[pl.ds(TAIL0, TAILN)],
                            out_hbm.at[pl.ds(cbase + lo + TAIL0, TAILN)])
        plsc.subcore_barrier()

    @pl.when(jnp.logical_and(c == 0, s == 0))
    def _():
        acc = jnp.zeros((16,), jnp.float32)
        for r in range(NS):
            pltpu.sync_copy(loss_sh.at[pl.ds(r * 16, 16)], lrow)
            acc = acc + lrow[...]
        lout[...] = acc * (LAMB / float(B * D))
        pl.delay(4096)
        pltpu.sync_copy(lout, loss_hbm)


_mesh = plsc.VectorSubcoreMesh(core_axis_name="c", subcore_axis_name="s",
                               num_cores=NC, num_subcores=NS)

_sc_call = pl.kernel(
    _body,
    out_type=(jax.ShapeDtypeStruct((16,), jnp.float32),
              jax.ShapeDtypeStruct((C, D), jnp.float32)),
    mesh=_mesh,
    compiler_params=pltpu.CompilerParams(use_tc_tiling_on_sc=False),
    scratch_types=(
        pltpu.VMEM((SPT,), jnp.int32),
        pltpu.VMEM((NBLK, BLK), jnp.int32),
        pltpu.VMEM((SPT,), jnp.int32),
        pltpu.VMEM((BLK,), jnp.float32),
        pltpu.VMEM((ZQ,), jnp.float32),
        pltpu.VMEM((PBLK, D), jnp.float32),
        pltpu.VMEM((PBLK, D), jnp.float32),
        pltpu.VMEM((PBLK,), jnp.float32),
        pltpu.VMEM((PBLK, D), jnp.float32),
        pltpu.VMEM((PBLK, D), jnp.float32),
        pltpu.VMEM((PBLK,), jnp.float32),
        pltpu.VMEM((SPT, D), jnp.float32),
        pltpu.VMEM((16,), jnp.float32),
        pltpu.VMEM((16,), jnp.float32),
        pltpu.MemorySpace.VMEM_SHARED((CNT_LEN,), jnp.float32),
        pltpu.MemorySpace.VMEM_SHARED((ACC_ROWS, D), jnp.float32),
        pltpu.MemorySpace.VMEM_SHARED((NS * 16,), jnp.float32),
        pltpu.SemaphoreType.DMA,
        pltpu.SemaphoreType.DMA,
        pltpu.SemaphoreType.DMA,
        pltpu.SemaphoreType.DMA,
        pltpu.SemaphoreType.DMA,
        pltpu.SemaphoreType.DMA,
        pltpu.SemaphoreType.DMA,
        pltpu.SemaphoreType.DMA,
    ),
)


@jax.jit
def kernel(features, targets, centers):
    loss_v, new_centers = _sc_call(features, targets, centers)
    return jnp.sum(loss_v), new_centers
